# Initial kernel scaffold; baseline (speedup 1.0000x reference)
import jax
import jax.numpy as jnp
from jax.experimental import pallas as pl


def kernel(tokens, embed_table):
    raise NotImplementedError("write your pallas kernel here")


# speedup vs baseline: 1.8416x; 1.8416x over previous
import jax
import jax.numpy as jnp
from jax import lax
from jax.experimental import pallas as pl
from jax.experimental.pallas import tpu as pltpu
from jax.experimental.pallas import tpu_sc as plsc

NUM_CORES = 2
NUM_SUBCORES = 16
NUM_WORKERS = NUM_CORES * NUM_SUBCORES


def kernel(tokens, embed_table):
    num_tokens = tokens.shape[0]
    dim = embed_table.shape[1]
    b_per_w = num_tokens // NUM_WORKERS

    mesh = plsc.VectorSubcoreMesh(core_axis_name="c", subcore_axis_name="s")

    @jax.jit
    def run(tok, table):
        @pl.kernel(
            mesh=mesh,
            out_type=jax.ShapeDtypeStruct((num_tokens, dim), table.dtype),
            scratch_types=[
                pltpu.VMEM((b_per_w,), jnp.int32),
                pltpu.VMEM((b_per_w, dim), table.dtype),
                pltpu.SemaphoreType.DMA,
            ],
        )
        def sc_gather(idx_hbm, table_hbm, out_hbm, idx_v, rows_v, sem):
            wid = lax.axis_index("s") * NUM_CORES + lax.axis_index("c")
            base = wid * b_per_w
            pltpu.sync_copy(idx_hbm.at[pl.ds(base, b_per_w)], idx_v)
            pltpu.async_copy(table_hbm.at[idx_v], rows_v, sem).wait()
            pltpu.sync_copy(rows_v, out_hbm.at[pl.ds(base, b_per_w)])

        return sc_gather(tok, table)

    return run(tokens.astype(jnp.int32), embed_table)
